# baseline (speedup 1.0000x reference)
import functools

import jax
import jax.numpy as jnp
from jax import lax
from jax.experimental import pallas as pl
from jax.experimental.pallas import tpu as pltpu
from jax.experimental.pallas import tpu_sc as plsc

RANK = 16
OUT_DIM = 64
NUM_CORES = 2
NUM_SUBCORES = 16
NW = NUM_CORES * NUM_SUBCORES


_PANEL = 1024


def _tc_pack_table(at):
    v = at.shape[1]
    steps = (v + _PANEL - 1) // _PANEL
    rows_out = steps * 128

    def body(at_ref, out_ref):
        ri = lax.broadcasted_iota(jnp.int32, (RANK, RANK), 0)
        ci = lax.broadcasted_iota(jnp.int32, (RANK, RANK), 1)
        eye = (ri == ci).astype(jnp.float32)
        t = lax.dot_general(
            at_ref[...], eye, (((0,), (0,)), ((), ())),
            preferred_element_type=jnp.float32,
        )
        pieces = [t[128 * k : 128 * (k + 1), :] for k in range(8)]
        out_ref[...] = jnp.concatenate(pieces, axis=1)

    return pl.pallas_call(
        body,
        grid=(steps,),
        in_specs=[pl.BlockSpec((RANK, _PANEL), lambda i: (0, i))],
        out_specs=pl.BlockSpec((128, 128), lambda i: (i, 0)),
        out_shape=jax.ShapeDtypeStruct((rows_out, 128), jnp.float32),
    )(at)


def _remap_idx(v):
    step = v >> 10
    rem = v & (_PANEL - 1)
    k = rem >> 7
    p = rem & 127
    return ((step << 7) + p) * 8 + k


def _sc_gather(idx, table, chunk):
    n = idx.shape[0]
    b_per_w = n // NW
    n_chunks = b_per_w // chunk
    mesh = plsc.VectorSubcoreMesh(core_axis_name="c", subcore_axis_name="s")

    pack = 128 // RANK

    @functools.partial(
        pl.kernel,
        mesh=mesh,
        out_type=jax.ShapeDtypeStruct((n // pack, 128), jnp.float32),
        scratch_types=[
            pltpu.VMEM((chunk,), jnp.int32),
            pltpu.VMEM((chunk, RANK), jnp.float32),
            pltpu.VMEM((chunk // pack, 128), jnp.float32),
            pltpu.SemaphoreType.DMA,
        ],
        compiler_params=pltpu.CompilerParams(use_tc_tiling_on_sc=False),
    )
    def k(idx_hbm, table_hbm, out_hbm, idx_v, rows_v, packed_v, sem):
        wid = lax.axis_index("s") * NUM_CORES + lax.axis_index("c")
        base = wid * b_per_w

        def body(i, carry):
            off = base + i * chunk
            pltpu.sync_copy(idx_hbm.at[pl.ds(off, chunk)], idx_v)
            pltpu.async_copy(table_hbm.at[idx_v], rows_v, sem).wait()

            def repack(j, c):
                for kk in range(pack):
                    packed_v[j, pl.ds(kk * RANK, RANK)] = rows_v[
                        j * pack + kk, :
                    ]
                return c

            lax.fori_loop(0, chunk // pack, repack, 0)
            pltpu.sync_copy(
                packed_v, out_hbm.at[pl.ds(off // pack, chunk // pack)]
            )
            return carry

        lax.fori_loop(0, n_chunks, body, 0)

    return k(idx, table)


def _tc_project_packed(emb_packed, w_packed):
    rows = emb_packed.shape[0]
    cols = 8 * OUT_DIM
    blk = 512

    def body(emb_ref, w_ref, out_ref):
        out_ref[...] = jnp.dot(
            emb_ref[...], w_ref[...], preferred_element_type=jnp.float32
        )

    return pl.pallas_call(
        body,
        grid=(rows // blk,),
        in_specs=[
            pl.BlockSpec((blk, 8 * RANK), lambda i: (i, 0)),
            pl.BlockSpec((8 * RANK, cols), lambda i: (0, 0)),
        ],
        out_specs=pl.BlockSpec((blk, cols), lambda i: (i, 0)),
        out_shape=jax.ShapeDtypeStruct((rows, cols), jnp.float32),
    )(emb_packed, w_packed)


def kernel(x, A, B):
    batch, fields = x.shape
    n = batch * fields
    idx = _remap_idx(x.reshape(n).astype(jnp.int32))
    table_packed = _tc_pack_table(A.T)
    table_rows = table_packed.shape[0] * (128 // RANK)
    emb_packed = _sc_gather(
        idx, table_packed.reshape(table_rows, RANK), chunk=1664
    )
    w_packed = (
        jnp.eye(8, dtype=jnp.float32)[:, None, :, None] * B[None, :, None, :]
    ).reshape(8 * RANK, 8 * OUT_DIM)
    outp = _tc_project_packed(emb_packed, w_packed)
    return outp.reshape(batch, fields, OUT_DIM)


# speedup vs baseline: 2.6390x; 2.6390x over previous
import functools

import jax
import jax.numpy as jnp
from jax import lax
from jax.experimental import pallas as pl
from jax.experimental.pallas import tpu as pltpu
from jax.experimental.pallas import tpu_sc as plsc

RANK = 16
OUT_DIM = 64
NUM_CORES = 2
NUM_SUBCORES = 16
NW = NUM_CORES * NUM_SUBCORES


_PANEL = 8192


def _tc_pack_table(at):
    v = at.shape[1]
    steps = (v + _PANEL - 1) // _PANEL
    blk = _PANEL // 8
    rows_out = steps * blk

    def body(at_ref, out_ref):
        ri = lax.broadcasted_iota(jnp.int32, (RANK, RANK), 0)
        ci = lax.broadcasted_iota(jnp.int32, (RANK, RANK), 1)
        eye = (ri == ci).astype(jnp.float32)
        t = lax.dot_general(
            at_ref[...], eye, (((0,), (0,)), ((), ())),
            preferred_element_type=jnp.float32,
        )
        pieces = [t[blk * k : blk * (k + 1), :] for k in range(8)]
        out_ref[...] = jnp.concatenate(pieces, axis=1)

    return pl.pallas_call(
        body,
        grid=(steps,),
        in_specs=[pl.BlockSpec((RANK, _PANEL), lambda i: (0, i))],
        out_specs=pl.BlockSpec((blk, 128), lambda i: (i, 0)),
        out_shape=jax.ShapeDtypeStruct((rows_out, 128), jnp.float32),
    )(at)


def _remap_idx(v):
    blk = _PANEL // 8
    step = v // _PANEL
    rem = v & (_PANEL - 1)
    k = rem // blk
    p = rem % blk
    return (step * blk + p) * 8 + k


def _sc_gather(idx, table, chunk):
    n = idx.shape[0]
    b_per_w = n // NW
    n_chunks = b_per_w // chunk
    mesh = plsc.VectorSubcoreMesh(core_axis_name="c", subcore_axis_name="s")

    pack = 128 // RANK

    @functools.partial(
        pl.kernel,
        mesh=mesh,
        out_type=jax.ShapeDtypeStruct((n // pack, 128), jnp.float32),
        scratch_types=[
            pltpu.VMEM((chunk,), jnp.int32),
            pltpu.VMEM((chunk, RANK), jnp.float32),
            pltpu.VMEM((chunk // pack, 128), jnp.float32),
            pltpu.SemaphoreType.DMA,
        ],
        compiler_params=pltpu.CompilerParams(use_tc_tiling_on_sc=False),
    )
    def k(idx_hbm, table_hbm, out_hbm, idx_v, rows_v, packed_v, sem):
        wid = lax.axis_index("s") * NUM_CORES + lax.axis_index("c")
        base = wid * b_per_w

        def body(i, carry):
            off = base + i * chunk
            pltpu.sync_copy(idx_hbm.at[pl.ds(off, chunk)], idx_v)
            pltpu.async_copy(table_hbm.at[idx_v], rows_v, sem).wait()

            def repack(j, c):
                for kk in range(pack):
                    packed_v[j, pl.ds(kk * RANK, RANK)] = rows_v[
                        j * pack + kk, :
                    ]
                return c

            lax.fori_loop(0, chunk // pack, repack, 0)
            pltpu.sync_copy(
                packed_v, out_hbm.at[pl.ds(off // pack, chunk // pack)]
            )
            return carry

        lax.fori_loop(0, n_chunks, body, 0)

    return k(idx, table)


def _tc_project_fields(emb_packed, proj, batch, fields):
    q8 = batch // 8

    def body(emb_ref, p_ref, out_ref):
        for k in range(8):
            s = emb_ref[:, RANK * k : RANK * (k + 1)]
            cols = lax.dot_general(
                p_ref[...],
                s,
                (((0,), (1,)), ((), ())),
                preferred_element_type=jnp.float32,
            )
            out_ref[0, :, k * q8 : (k + 1) * q8] = cols

    return pl.pallas_call(
        body,
        grid=(fields,),
        in_specs=[
            pl.BlockSpec((q8, 128), lambda f: (f, 0)),
            pl.BlockSpec((RANK, OUT_DIM), lambda f: (0, 0)),
        ],
        out_specs=pl.BlockSpec((1, OUT_DIM, batch), lambda f: (f, 0, 0)),
        out_shape=jax.ShapeDtypeStruct((fields, OUT_DIM, batch), jnp.float32),
    )(emb_packed, proj)


def kernel(x, A, B):
    batch, fields = x.shape
    n = batch * fields
    idx = _remap_idx(
        x.T.astype(jnp.int32)
        .reshape(fields, 8, batch // 8)
        .transpose(0, 2, 1)
        .reshape(n)
    )
    table_packed = _tc_pack_table(A.T)
    table_rows = table_packed.shape[0] * (128 // RANK)
    emb_packed = _sc_gather(
        idx, table_packed.reshape(table_rows, RANK), chunk=1664
    )
    out_t = _tc_project_fields(emb_packed, B, batch, fields)
    return out_t.transpose(2, 0, 1)


# speedup vs baseline: 2.6657x; 1.0101x over previous
import functools

import jax
import jax.numpy as jnp
from jax import lax
from jax.experimental import pallas as pl
from jax.experimental.pallas import tpu as pltpu
from jax.experimental.pallas import tpu_sc as plsc

RANK = 16
OUT_DIM = 64
NUM_CORES = 2
NUM_SUBCORES = 16
NW = NUM_CORES * NUM_SUBCORES


_PANEL = 32768


def _tc_pack_table(at):
    v = at.shape[1]
    steps = (v + _PANEL - 1) // _PANEL
    blk = _PANEL // 8
    rows_out = steps * blk

    def body(at_ref, out_ref):
        ri = lax.broadcasted_iota(jnp.int32, (RANK, RANK), 0)
        ci = lax.broadcasted_iota(jnp.int32, (RANK, RANK), 1)
        eye = (ri == ci).astype(jnp.float32)
        t = lax.dot_general(
            at_ref[...], eye, (((0,), (0,)), ((), ())),
            preferred_element_type=jnp.float32,
        )
        pieces = [t[blk * k : blk * (k + 1), :] for k in range(8)]
        out_ref[...] = jnp.concatenate(pieces, axis=1)

    return pl.pallas_call(
        body,
        grid=(steps,),
        in_specs=[pl.BlockSpec((RANK, _PANEL), lambda i: (0, i))],
        out_specs=pl.BlockSpec((blk, 128), lambda i: (i, 0)),
        out_shape=jax.ShapeDtypeStruct((rows_out, 128), jnp.float32),
    )(at)


def _remap_idx(v):
    blk = _PANEL // 8
    step = v // _PANEL
    rem = v & (_PANEL - 1)
    k = rem // blk
    p = rem % blk
    return (step * blk + p) * 8 + k


def _sc_gather(idx, table, chunk):
    n = idx.shape[0]
    b_per_w = n // NW
    n_chunks = b_per_w // chunk
    mesh = plsc.VectorSubcoreMesh(core_axis_name="c", subcore_axis_name="s")

    pack = 128 // RANK

    @functools.partial(
        pl.kernel,
        mesh=mesh,
        out_type=jax.ShapeDtypeStruct((n // pack, 128), jnp.float32),
        scratch_types=[
            pltpu.VMEM((chunk,), jnp.int32),
            pltpu.VMEM((chunk, RANK), jnp.float32),
            pltpu.VMEM((chunk // pack, 128), jnp.float32),
            pltpu.SemaphoreType.DMA,
        ],
        compiler_params=pltpu.CompilerParams(use_tc_tiling_on_sc=False),
    )
    def k(idx_hbm, table_hbm, out_hbm, idx_v, rows_v, packed_v, sem):
        wid = lax.axis_index("s") * NUM_CORES + lax.axis_index("c")
        base = wid * b_per_w

        def body(i, carry):
            off = base + i * chunk
            pltpu.sync_copy(idx_hbm.at[pl.ds(off, chunk)], idx_v)
            pltpu.async_copy(table_hbm.at[idx_v], rows_v, sem).wait()

            def repack(j, c):
                for kk in range(pack):
                    packed_v[j, pl.ds(kk * RANK, RANK)] = rows_v[
                        j * pack + kk, :
                    ]
                return c

            lax.fori_loop(0, chunk // pack, repack, 0)
            pltpu.sync_copy(
                packed_v, out_hbm.at[pl.ds(off // pack, chunk // pack)]
            )
            return carry

        lax.fori_loop(0, n_chunks, body, 0)

    return k(idx, table)


def _tc_project_fields(emb_packed, proj, batch, fields):
    q8 = batch // 8

    def body(emb_ref, p_ref, out_ref):
        for k in range(8):
            s = emb_ref[:, RANK * k : RANK * (k + 1)]
            cols = lax.dot_general(
                p_ref[...],
                s,
                (((0,), (1,)), ((), ())),
                preferred_element_type=jnp.float32,
            )
            out_ref[0, :, k * q8 : (k + 1) * q8] = cols

    return pl.pallas_call(
        body,
        grid=(fields,),
        in_specs=[
            pl.BlockSpec((q8, 128), lambda f: (f, 0)),
            pl.BlockSpec((RANK, OUT_DIM), lambda f: (0, 0)),
        ],
        out_specs=pl.BlockSpec((1, OUT_DIM, batch), lambda f: (f, 0, 0)),
        out_shape=jax.ShapeDtypeStruct((fields, OUT_DIM, batch), jnp.float32),
    )(emb_packed, proj)


def kernel(x, A, B):
    batch, fields = x.shape
    n = batch * fields
    idx = _remap_idx(
        x.T.astype(jnp.int32)
        .reshape(fields, 8, batch // 8)
        .transpose(0, 2, 1)
        .reshape(n)
    )
    table_packed = _tc_pack_table(A.T)
    table_rows = table_packed.shape[0] * (128 // RANK)
    emb_packed = _sc_gather(
        idx, table_packed.reshape(table_rows, RANK), chunk=1664
    )
    out_t = _tc_project_fields(emb_packed, B, batch, fields)
    return out_t.transpose(2, 0, 1)
